# Initial kernel scaffold; baseline (speedup 1.0000x reference)
import jax
import jax.numpy as jnp
from jax import lax
from jax.experimental import pallas as pl
from jax.experimental.pallas import tpu as pltpu
from jax.experimental.pallas import tpu_sc as plsc

N = 10000
E = 320000
DIN = 128
H = 64
DE = 16
DEC = 32

NC = 2
NS = 16
CH = 125
EPT = E // (NC * NS)
NCH = EPT // CH
ROWS_PT = N // NS
IDX_ROWS = E // CH
IDX_PER_SC = IDX_ROWS // NC

BLK = 2000
BLE = 4000

_MESH = plsc.VectorSubcoreMesh(core_axis_name="c", subcore_axis_name="s")


def _deg_body(dst2d, zn, ones_h, out, d_v, ones_v, acc):
    c = lax.axis_index("c")
    s = lax.axis_index("s")
    pltpu.sync_copy(ones_h, ones_v)

    @pl.when(s == 0)
    def _zero():
        pltpu.sync_copy(zn, acc)

    plsc.subcore_barrier()
    pltpu.sync_copy(dst2d.at[pl.ds(c * IDX_PER_SC + s * NCH, NCH)], d_v)

    def body(j, carry):
        pltpu.sync_copy(ones_v, acc.at[d_v.at[j]], add=True)
        return carry

    lax.fori_loop(0, NCH, body, 0)
    plsc.subcore_barrier()

    @pl.when(s == 0)
    def _out():
        pltpu.sync_copy(acc, out.at[pl.ds(c * N, N)])


_deg_call = pl.kernel(
    _deg_body,
    out_type=jax.ShapeDtypeStruct((2 * N,), jnp.float32),
    mesh=_MESH,
    scratch_types=[
        pltpu.VMEM((NCH, CH), jnp.int32),
        pltpu.VMEM((CH,), jnp.float32),
        pltpu.VMEM_SHARED((N,), jnp.float32),
    ],
)


def _agg_body(hw, src2d, dst2d, z2d, out_a, out_b, s_v, d_v, rows, acc, gsem):
    c = lax.axis_index("c")
    s = lax.axis_index("s")
    pltpu.sync_copy(z2d.at[pl.ds(s * ROWS_PT, ROWS_PT)],
                    acc.at[pl.ds(s * ROWS_PT, ROWS_PT)])
    plsc.subcore_barrier()
    base = c * IDX_PER_SC + s * NCH
    pltpu.sync_copy(src2d.at[pl.ds(base, NCH)], s_v)
    pltpu.sync_copy(dst2d.at[pl.ds(base, NCH)], d_v)

    def body(j, carry):
        pltpu.async_copy(hw.at[s_v.at[j]], rows, gsem).wait()
        pltpu.sync_copy(rows, acc.at[d_v.at[j]], add=True)
        return carry

    lax.fori_loop(0, NCH, body, 0)
    plsc.subcore_barrier()

    @pl.when(c == 0)
    def _out_a():
        pltpu.sync_copy(acc.at[pl.ds(s * ROWS_PT, ROWS_PT)],
                        out_a.at[pl.ds(s * ROWS_PT, ROWS_PT)])

    @pl.when(c == 1)
    def _out_b():
        pltpu.sync_copy(acc.at[pl.ds(s * ROWS_PT, ROWS_PT)],
                        out_b.at[pl.ds(s * ROWS_PT, ROWS_PT)])


_agg_call = pl.kernel(
    _agg_body,
    out_type=[jax.ShapeDtypeStruct((N, H), jnp.float32),
              jax.ShapeDtypeStruct((N, H), jnp.float32)],
    mesh=_MESH,
    scratch_types=[
        pltpu.VMEM((NCH, CH), jnp.int32),
        pltpu.VMEM((NCH, CH), jnp.int32),
        pltpu.VMEM((CH, H), jnp.float32),
        pltpu.VMEM_SHARED((N, H), jnp.float32),
        pltpu.SemaphoreType.DMA,
    ],
)


def _dec_body(p, q, src2d, dst2d, g1, g2, s_v, d_v, rp, rq, sp, sq):
    c = lax.axis_index("c")
    s = lax.axis_index("s")
    base = c * IDX_PER_SC + s * NCH
    pltpu.sync_copy(src2d.at[pl.ds(base, NCH)], s_v)
    pltpu.sync_copy(dst2d.at[pl.ds(base, NCH)], d_v)
    ebase = base * CH

    def body(j, carry):
        cp = pltpu.async_copy(p.at[s_v.at[j]], rp, sp)
        cq = pltpu.async_copy(q.at[d_v.at[j]], rq, sq)
        cp.wait()
        pltpu.sync_copy(rp, g1.at[pl.ds(ebase + j * CH, CH)])
        cq.wait()
        pltpu.sync_copy(rq, g2.at[pl.ds(ebase + j * CH, CH)])
        return carry

    lax.fori_loop(0, NCH, body, 0)


_dec_call = pl.kernel(
    _dec_body,
    out_type=[jax.ShapeDtypeStruct((E, DEC), jnp.float32),
              jax.ShapeDtypeStruct((E, DEC), jnp.float32)],
    mesh=_MESH,
    scratch_types=[
        pltpu.VMEM((NCH, CH), jnp.int32),
        pltpu.VMEM((NCH, CH), jnp.int32),
        pltpu.VMEM((CH, DEC), jnp.float32),
        pltpu.VMEM((CH, DEC), jnp.float32),
        pltpu.SemaphoreType.DMA,
        pltpu.SemaphoreType.DMA,
    ],
)


def _tc1_body(x_r, w0_r, dega_r, degb_r, hws_r, dinv_r):
    deg = dega_r[...] + degb_r[...] + 1.0
    dinv = lax.rsqrt(deg)
    hw = jnp.dot(x_r[...], w0_r[...], preferred_element_type=jnp.float32)
    hws_r[...] = hw * dinv
    dinv_r[...] = dinv


def _tc1(x, w0, dega, degb):
    return pl.pallas_call(
        _tc1_body,
        grid=(N // BLK,),
        in_specs=[
            pl.BlockSpec((BLK, DIN), lambda i: (i, 0)),
            pl.BlockSpec((DIN, H), lambda i: (0, 0)),
            pl.BlockSpec((BLK, 1), lambda i: (i, 0)),
            pl.BlockSpec((BLK, 1), lambda i: (i, 0)),
        ],
        out_specs=[
            pl.BlockSpec((BLK, H), lambda i: (i, 0)),
            pl.BlockSpec((BLK, 1), lambda i: (i, 0)),
        ],
        out_shape=[
            jax.ShapeDtypeStruct((N, H), jnp.float32),
            jax.ShapeDtypeStruct((N, 1), jnp.float32),
        ],
    )(x, w0, dega, degb)


def _prologue(agg_a, agg_b, hwp, dinv, b, g, be):
    pre = dinv * (agg_a + agg_b + hwp) + b
    mu = jnp.mean(pre, axis=-1, keepdims=True)
    d = pre - mu
    var = jnp.mean(d * d, axis=-1, keepdims=True)
    hn = d * lax.rsqrt(var + 1e-5) * g + be
    return jnp.maximum(hn, 0.0)


def _tc2_body(agg_a_r, agg_b_r, hwp_r, dinv_r, w_r, b_r, g_r, be_r, out_r):
    h = _prologue(agg_a_r[...], agg_b_r[...], hwp_r[...], dinv_r[...],
                  b_r[...], g_r[...], be_r[...])
    out_r[...] = jnp.dot(h, w_r[...],
                         preferred_element_type=jnp.float32) * dinv_r[...]


def _tc2(agg_a, agg_b, hwp, dinv, w, b, g, be):
    return pl.pallas_call(
        _tc2_body,
        grid=(N // BLK,),
        in_specs=[
            pl.BlockSpec((BLK, H), lambda i: (i, 0)),
            pl.BlockSpec((BLK, H), lambda i: (i, 0)),
            pl.BlockSpec((BLK, H), lambda i: (i, 0)),
            pl.BlockSpec((BLK, 1), lambda i: (i, 0)),
            pl.BlockSpec((H, H), lambda i: (0, 0)),
            pl.BlockSpec((1, H), lambda i: (0, 0)),
            pl.BlockSpec((1, H), lambda i: (0, 0)),
            pl.BlockSpec((1, H), lambda i: (0, 0)),
        ],
        out_specs=pl.BlockSpec((BLK, H), lambda i: (i, 0)),
        out_shape=jax.ShapeDtypeStruct((N, H), jnp.float32),
    )(agg_a, agg_b, hwp, dinv, w, b, g, be)


def _tc3_body(agg_a_r, agg_b_r, hwp_r, dinv_r, w1a_r, w1b_r, b_r, g_r, be_r,
              p_r, q_r):
    h = _prologue(agg_a_r[...], agg_b_r[...], hwp_r[...], dinv_r[...],
                  b_r[...], g_r[...], be_r[...])
    p_r[...] = jnp.dot(h, w1a_r[...], preferred_element_type=jnp.float32)
    q_r[...] = jnp.dot(h, w1b_r[...], preferred_element_type=jnp.float32)


def _tc3(agg_a, agg_b, hwp, dinv, w1a, w1b, b, g, be):
    return pl.pallas_call(
        _tc3_body,
        grid=(N // BLK,),
        in_specs=[
            pl.BlockSpec((BLK, H), lambda i: (i, 0)),
            pl.BlockSpec((BLK, H), lambda i: (i, 0)),
            pl.BlockSpec((BLK, H), lambda i: (i, 0)),
            pl.BlockSpec((BLK, 1), lambda i: (i, 0)),
            pl.BlockSpec((H, DEC), lambda i: (0, 0)),
            pl.BlockSpec((H, DEC), lambda i: (0, 0)),
            pl.BlockSpec((1, H), lambda i: (0, 0)),
            pl.BlockSpec((1, H), lambda i: (0, 0)),
            pl.BlockSpec((1, H), lambda i: (0, 0)),
        ],
        out_specs=[
            pl.BlockSpec((BLK, DEC), lambda i: (i, 0)),
            pl.BlockSpec((BLK, DEC), lambda i: (i, 0)),
        ],
        out_shape=[
            jax.ShapeDtypeStruct((N, DEC), jnp.float32),
            jax.ShapeDtypeStruct((N, DEC), jnp.float32),
        ],
    )(agg_a, agg_b, hwp, dinv, w1a, w1b, b, g, be)


def _tc4_body(g1_r, g2_r, ea_r, w1c_r, db1_r, w2_r, db2_r, w3_r, db3_r,
              out_r):
    r = jnp.dot(ea_r[...], w1c_r[...], preferred_element_type=jnp.float32)
    z = jnp.maximum(g1_r[...] + g2_r[...] + r + db1_r[...], 0.0)
    z2 = jnp.maximum(
        jnp.dot(z, w2_r[...], preferred_element_type=jnp.float32) + db2_r[...],
        0.0)
    out_r[...] = jnp.sum(z2 * w3_r[...], axis=-1, keepdims=True) + db3_r[...]


def _tc4(g1, g2, ea, w1c, db1, w2, db2, w3t, db3):
    return pl.pallas_call(
        _tc4_body,
        grid=(E // BLE,),
        in_specs=[
            pl.BlockSpec((BLE, DEC), lambda i: (i, 0)),
            pl.BlockSpec((BLE, DEC), lambda i: (i, 0)),
            pl.BlockSpec((BLE, DE), lambda i: (i, 0)),
            pl.BlockSpec((DE, DEC), lambda i: (0, 0)),
            pl.BlockSpec((1, DEC), lambda i: (0, 0)),
            pl.BlockSpec((DEC, DEC // 2), lambda i: (0, 0)),
            pl.BlockSpec((1, DEC // 2), lambda i: (0, 0)),
            pl.BlockSpec((1, DEC // 2), lambda i: (0, 0)),
            pl.BlockSpec((1, 1), lambda i: (0, 0)),
        ],
        out_specs=pl.BlockSpec((BLE, 1), lambda i: (i, 0)),
        out_shape=jax.ShapeDtypeStruct((E, 1), jnp.float32),
    )(g1, g2, ea, w1c, db1, w2, db2, w3t, db3)


def kernel(x, edge_index, edge_attr, W0, b0, W1, b1, W2, b2,
           g0, be0, g1, be1, g2, be2,
           dW1, db1, dW2, db2, dW3, db3):
    f32 = jnp.float32
    src = edge_index[0].astype(jnp.int32)
    dst = edge_index[1].astype(jnp.int32)
    src2d = src.reshape(IDX_ROWS, CH)
    dst2d = dst.reshape(IDX_ROWS, CH)
    zn = jnp.zeros((N,), f32)
    z2d = jnp.zeros((N, H), f32)
    ones_h = jnp.ones((CH,), f32)

    deg2 = _deg_call(dst2d, zn, ones_h)
    dega = deg2[:N].reshape(N, 1)
    degb = deg2[N:].reshape(N, 1)

    hws0, dinv = _tc1(x, W0, dega, degb)

    agg_a, agg_b = _agg_call(hws0, src2d, dst2d, z2d)
    hws1 = _tc2(agg_a, agg_b, hws0, dinv, W1,
                b0.reshape(1, H), g0.reshape(1, H), be0.reshape(1, H))

    agg_a, agg_b = _agg_call(hws1, src2d, dst2d, z2d)
    hws2 = _tc2(agg_a, agg_b, hws1, dinv, W2,
                b1.reshape(1, H), g1.reshape(1, H), be1.reshape(1, H))

    agg_a, agg_b = _agg_call(hws2, src2d, dst2d, z2d)
    p, q = _tc3(agg_a, agg_b, hws2, dinv, dW1[:H], dW1[H:2 * H],
                b2.reshape(1, H), g2.reshape(1, H), be2.reshape(1, H))

    g1e, g2e = _dec_call(p, q, src2d, dst2d)
    out2d = _tc4(g1e, g2e, edge_attr, dW1[2 * H:], db1.reshape(1, DEC),
                 dW2, db2.reshape(1, DEC // 2), dW3.reshape(1, DEC // 2),
                 db3.reshape(1, 1))
    return out2d[:, 0]


# speedup vs baseline: 9.4622x; 9.4622x over previous
import jax
import jax.numpy as jnp
from jax import lax
from jax.experimental import pallas as pl
from jax.experimental.pallas import tpu as pltpu
from jax.experimental.pallas import tpu_sc as plsc

N = 10000
E = 320000
DIN = 128
H = 64
DE = 16
DEC = 32

NC = 2
NS = 16
CH = 128
EPT = E // (NC * NS)
NCHF = EPT // CH
TAIL = EPT - NCHF * CH
RZ = 1000
NZT = N // RZ

BLK = 2000
BLE = 4000

_MESH = plsc.VectorSubcoreMesh(core_axis_name="c", subcore_axis_name="s",
                               num_cores=NC, num_subcores=NS)


DW = 8


def _deg_body(dst1, zn, ones_h, out, d_v, d_t, ones_v, acc):
    c = lax.axis_index("c")
    s = lax.axis_index("s")
    w = c * NS + s
    ebase = w * EPT
    pltpu.sync_copy(ones_h, ones_v)

    @pl.when(s < NZT)
    def _zero():
        pltpu.sync_copy(zn.at[pl.ds(s * RZ, RZ)], acc.at[pl.ds(s * RZ, RZ)])

    plsc.subcore_barrier()

    def body(j, carry):
        pltpu.sync_copy(dst1.at[pl.ds(ebase + j * CH, CH)], d_v)
        pltpu.sync_copy(ones_v.at[pl.ds(0, CH)], acc.at[d_v], add=True)
        return carry

    lax.fori_loop(0, NCHF, body, 0)
    pltpu.sync_copy(dst1.at[pl.ds(ebase + NCHF * CH, TAIL)], d_t)
    pltpu.sync_copy(ones_v.at[pl.ds(0, TAIL)], acc.at[d_t], add=True)
    plsc.subcore_barrier()

    @pl.when(s < NZT)
    def _out():
        pltpu.sync_copy(acc.at[pl.ds(s * RZ, RZ)],
                        out.at[pl.ds(c * N + s * RZ, RZ)])


_deg_call = pl.kernel(
    _deg_body,
    out_type=jax.ShapeDtypeStruct((2 * N, DW), jnp.float32),
    mesh=_MESH,
    compiler_params=pltpu.CompilerParams(use_tc_tiling_on_sc=False),
    scratch_types=[
        pltpu.VMEM((CH,), jnp.int32),
        pltpu.VMEM((TAIL,), jnp.int32),
        pltpu.VMEM((CH, DW), jnp.float32),
        pltpu.VMEM_SHARED((N, DW), jnp.float32),
    ],
)


def _agg_body(hw, src1, dst1, z2d, out_a, out_b,
              s_v, d_v, s_t, d_t, rows, rows_t, acc, gsem):
    c = lax.axis_index("c")
    s = lax.axis_index("s")
    w = c * NS + s
    ebase = w * EPT

    @pl.when(s < NZT)
    def _zero():
        pltpu.sync_copy(z2d.at[pl.ds(s * RZ, RZ)], acc.at[pl.ds(s * RZ, RZ)])

    plsc.subcore_barrier()

    def body(j, carry):
        pltpu.sync_copy(src1.at[pl.ds(ebase + j * CH, CH)], s_v)
        pltpu.sync_copy(dst1.at[pl.ds(ebase + j * CH, CH)], d_v)
        pltpu.async_copy(hw.at[s_v], rows, gsem).wait()
        pltpu.sync_copy(rows, acc.at[d_v], add=True)
        return carry

    lax.fori_loop(0, NCHF, body, 0)
    pltpu.sync_copy(src1.at[pl.ds(ebase + NCHF * CH, TAIL)], s_t)
    pltpu.sync_copy(dst1.at[pl.ds(ebase + NCHF * CH, TAIL)], d_t)
    pltpu.async_copy(hw.at[s_t], rows_t, gsem).wait()
    pltpu.sync_copy(rows_t, acc.at[d_t], add=True)
    plsc.subcore_barrier()

    @pl.when((s < NZT) & (c == 0))
    def _out_a():
        pltpu.sync_copy(acc.at[pl.ds(s * RZ, RZ)],
                        out_a.at[pl.ds(s * RZ, RZ)])

    @pl.when((s < NZT) & (c == 1))
    def _out_b():
        pltpu.sync_copy(acc.at[pl.ds(s * RZ, RZ)],
                        out_b.at[pl.ds(s * RZ, RZ)])


_agg_call = pl.kernel(
    _agg_body,
    out_type=[jax.ShapeDtypeStruct((N, H), jnp.float32),
              jax.ShapeDtypeStruct((N, H), jnp.float32)],
    mesh=_MESH,
    compiler_params=pltpu.CompilerParams(use_tc_tiling_on_sc=False),
    scratch_types=[
        pltpu.VMEM((CH,), jnp.int32),
        pltpu.VMEM((CH,), jnp.int32),
        pltpu.VMEM((TAIL,), jnp.int32),
        pltpu.VMEM((TAIL,), jnp.int32),
        pltpu.VMEM((CH, H), jnp.float32),
        pltpu.VMEM((TAIL, H), jnp.float32),
        pltpu.VMEM_SHARED((N, H), jnp.float32),
        pltpu.SemaphoreType.DMA,
    ],
)


def _dec_body(p, q, src1, dst1, g1, g2,
              s_v, d_v, s_t, d_t, rp, rq, rp_t, rq_t, sp, sq):
    c = lax.axis_index("c")
    s = lax.axis_index("s")
    w = c * NS + s
    ebase = w * EPT

    def body(j, carry):
        pltpu.sync_copy(src1.at[pl.ds(ebase + j * CH, CH)], s_v)
        pltpu.sync_copy(dst1.at[pl.ds(ebase + j * CH, CH)], d_v)
        cp = pltpu.async_copy(p.at[s_v], rp, sp)
        cq = pltpu.async_copy(q.at[d_v], rq, sq)
        cp.wait()
        pltpu.sync_copy(rp, g1.at[pl.ds(ebase + j * CH, CH)])
        cq.wait()
        pltpu.sync_copy(rq, g2.at[pl.ds(ebase + j * CH, CH)])
        return carry

    lax.fori_loop(0, NCHF, body, 0)
    pltpu.sync_copy(src1.at[pl.ds(ebase + NCHF * CH, TAIL)], s_t)
    pltpu.sync_copy(dst1.at[pl.ds(ebase + NCHF * CH, TAIL)], d_t)
    cp = pltpu.async_copy(p.at[s_t], rp_t, sp)
    cq = pltpu.async_copy(q.at[d_t], rq_t, sq)
    cp.wait()
    pltpu.sync_copy(rp_t, g1.at[pl.ds(ebase + NCHF * CH, TAIL)])
    cq.wait()
    pltpu.sync_copy(rq_t, g2.at[pl.ds(ebase + NCHF * CH, TAIL)])


_dec_call = pl.kernel(
    _dec_body,
    out_type=[jax.ShapeDtypeStruct((E, DEC), jnp.float32),
              jax.ShapeDtypeStruct((E, DEC), jnp.float32)],
    mesh=_MESH,
    compiler_params=pltpu.CompilerParams(use_tc_tiling_on_sc=False),
    scratch_types=[
        pltpu.VMEM((CH,), jnp.int32),
        pltpu.VMEM((CH,), jnp.int32),
        pltpu.VMEM((TAIL,), jnp.int32),
        pltpu.VMEM((TAIL,), jnp.int32),
        pltpu.VMEM((CH, DEC), jnp.float32),
        pltpu.VMEM((CH, DEC), jnp.float32),
        pltpu.VMEM((TAIL, DEC), jnp.float32),
        pltpu.VMEM((TAIL, DEC), jnp.float32),
        pltpu.SemaphoreType.DMA,
        pltpu.SemaphoreType.DMA,
    ],
)


def _tc1_body(x_r, w0_r, dega_r, degb_r, hws_r, dinv_r):
    deg = dega_r[...] + degb_r[...] + 1.0
    dinv = lax.rsqrt(deg)
    hw = jnp.dot(x_r[...], w0_r[...], preferred_element_type=jnp.float32)
    hws_r[...] = hw * dinv
    dinv_r[...] = dinv


def _tc1(x, w0, dega, degb):
    return pl.pallas_call(
        _tc1_body,
        grid=(N // BLK,),
        in_specs=[
            pl.BlockSpec((BLK, DIN), lambda i: (i, 0)),
            pl.BlockSpec((DIN, H), lambda i: (0, 0)),
            pl.BlockSpec((BLK, 1), lambda i: (i, 0)),
            pl.BlockSpec((BLK, 1), lambda i: (i, 0)),
        ],
        out_specs=[
            pl.BlockSpec((BLK, H), lambda i: (i, 0)),
            pl.BlockSpec((BLK, 1), lambda i: (i, 0)),
        ],
        out_shape=[
            jax.ShapeDtypeStruct((N, H), jnp.float32),
            jax.ShapeDtypeStruct((N, 1), jnp.float32),
        ],
    )(x, w0, dega, degb)


def _prologue(agg_a, agg_b, hwp, dinv, b, g, be):
    pre = dinv * (agg_a + agg_b + hwp) + b
    mu = jnp.mean(pre, axis=-1, keepdims=True)
    d = pre - mu
    var = jnp.mean(d * d, axis=-1, keepdims=True)
    hn = d * lax.rsqrt(var + 1e-5) * g + be
    return jnp.maximum(hn, 0.0)


def _tc2_body(agg_a_r, agg_b_r, hwp_r, dinv_r, w_r, b_r, g_r, be_r, out_r):
    h = _prologue(agg_a_r[...], agg_b_r[...], hwp_r[...], dinv_r[...],
                  b_r[...], g_r[...], be_r[...])
    out_r[...] = jnp.dot(h, w_r[...],
                         preferred_element_type=jnp.float32) * dinv_r[...]


def _tc2(agg_a, agg_b, hwp, dinv, w, b, g, be):
    return pl.pallas_call(
        _tc2_body,
        grid=(N // BLK,),
        in_specs=[
            pl.BlockSpec((BLK, H), lambda i: (i, 0)),
            pl.BlockSpec((BLK, H), lambda i: (i, 0)),
            pl.BlockSpec((BLK, H), lambda i: (i, 0)),
            pl.BlockSpec((BLK, 1), lambda i: (i, 0)),
            pl.BlockSpec((H, H), lambda i: (0, 0)),
            pl.BlockSpec((1, H), lambda i: (0, 0)),
            pl.BlockSpec((1, H), lambda i: (0, 0)),
            pl.BlockSpec((1, H), lambda i: (0, 0)),
        ],
        out_specs=pl.BlockSpec((BLK, H), lambda i: (i, 0)),
        out_shape=jax.ShapeDtypeStruct((N, H), jnp.float32),
    )(agg_a, agg_b, hwp, dinv, w, b, g, be)


def _tc3_body(agg_a_r, agg_b_r, hwp_r, dinv_r, w1a_r, w1b_r, b_r, g_r, be_r,
              p_r, q_r):
    h = _prologue(agg_a_r[...], agg_b_r[...], hwp_r[...], dinv_r[...],
                  b_r[...], g_r[...], be_r[...])
    p_r[...] = jnp.dot(h, w1a_r[...], preferred_element_type=jnp.float32)
    q_r[...] = jnp.dot(h, w1b_r[...], preferred_element_type=jnp.float32)


def _tc3(agg_a, agg_b, hwp, dinv, w1a, w1b, b, g, be):
    return pl.pallas_call(
        _tc3_body,
        grid=(N // BLK,),
        in_specs=[
            pl.BlockSpec((BLK, H), lambda i: (i, 0)),
            pl.BlockSpec((BLK, H), lambda i: (i, 0)),
            pl.BlockSpec((BLK, H), lambda i: (i, 0)),
            pl.BlockSpec((BLK, 1), lambda i: (i, 0)),
            pl.BlockSpec((H, DEC), lambda i: (0, 0)),
            pl.BlockSpec((H, DEC), lambda i: (0, 0)),
            pl.BlockSpec((1, H), lambda i: (0, 0)),
            pl.BlockSpec((1, H), lambda i: (0, 0)),
            pl.BlockSpec((1, H), lambda i: (0, 0)),
        ],
        out_specs=[
            pl.BlockSpec((BLK, DEC), lambda i: (i, 0)),
            pl.BlockSpec((BLK, DEC), lambda i: (i, 0)),
        ],
        out_shape=[
            jax.ShapeDtypeStruct((N, DEC), jnp.float32),
            jax.ShapeDtypeStruct((N, DEC), jnp.float32),
        ],
    )(agg_a, agg_b, hwp, dinv, w1a, w1b, b, g, be)


def _tc4_body(g1_r, g2_r, ea_r, w1c_r, db1_r, w2_r, db2_r, w3_r, db3_r,
              out_r):
    r = jnp.dot(ea_r[...], w1c_r[...], preferred_element_type=jnp.float32)
    z = jnp.maximum(g1_r[...] + g2_r[...] + r + db1_r[...], 0.0)
    z2 = jnp.maximum(
        jnp.dot(z, w2_r[...], preferred_element_type=jnp.float32) + db2_r[...],
        0.0)
    out_r[...] = jnp.sum(z2 * w3_r[...], axis=-1, keepdims=True) + db3_r[...]


def _tc4(g1, g2, ea, w1c, db1, w2, db2, w3t, db3):
    return pl.pallas_call(
        _tc4_body,
        grid=(E // BLE,),
        in_specs=[
            pl.BlockSpec((BLE, DEC), lambda i: (i, 0)),
            pl.BlockSpec((BLE, DEC), lambda i: (i, 0)),
            pl.BlockSpec((BLE, DE), lambda i: (i, 0)),
            pl.BlockSpec((DE, DEC), lambda i: (0, 0)),
            pl.BlockSpec((1, DEC), lambda i: (0, 0)),
            pl.BlockSpec((DEC, DEC // 2), lambda i: (0, 0)),
            pl.BlockSpec((1, DEC // 2), lambda i: (0, 0)),
            pl.BlockSpec((1, DEC // 2), lambda i: (0, 0)),
            pl.BlockSpec((1, 1), lambda i: (0, 0)),
        ],
        out_specs=pl.BlockSpec((BLE, 1), lambda i: (i, 0)),
        out_shape=jax.ShapeDtypeStruct((E, 1), jnp.float32),
    )(g1, g2, ea, w1c, db1, w2, db2, w3t, db3)


def kernel(x, edge_index, edge_attr, W0, b0, W1, b1, W2, b2,
           g0, be0, g1, be1, g2, be2,
           dW1, db1, dW2, db2, dW3, db3):
    f32 = jnp.float32
    src = edge_index[0].astype(jnp.int32)
    dst = edge_index[1].astype(jnp.int32)
    zn = jnp.zeros((N, DW), f32)
    z2d = jnp.zeros((N, H), f32)
    ones_h = jnp.ones((CH, DW), f32)

    deg2 = _deg_call(dst, zn, ones_h)
    dega = deg2[:N, :1]
    degb = deg2[N:, :1]

    hws0, dinv = _tc1(x, W0, dega, degb)

    agg_a, agg_b = _agg_call(hws0, src, dst, z2d)
    hws1 = _tc2(agg_a, agg_b, hws0, dinv, W1,
                b0.reshape(1, H), g0.reshape(1, H), be0.reshape(1, H))

    agg_a, agg_b = _agg_call(hws1, src, dst, z2d)
    hws2 = _tc2(agg_a, agg_b, hws1, dinv, W2,
                b1.reshape(1, H), g1.reshape(1, H), be1.reshape(1, H))

    agg_a, agg_b = _agg_call(hws2, src, dst, z2d)
    p, q = _tc3(agg_a, agg_b, hws2, dinv, dW1[:H], dW1[H:2 * H],
                b2.reshape(1, H), g2.reshape(1, H), be2.reshape(1, H))

    g1e, g2e = _dec_call(p, q, src, dst)
    out2d = _tc4(g1e, g2e, edge_attr, dW1[2 * H:], db1.reshape(1, DEC),
                 dW2, db2.reshape(1, DEC // 2), dW3.reshape(1, DEC // 2),
                 db3.reshape(1, 1))
    return out2d[:, 0]


# speedup vs baseline: 12.7012x; 1.3423x over previous
import jax
import jax.numpy as jnp
from jax import lax
from jax.experimental import pallas as pl
from jax.experimental.pallas import tpu as pltpu
from jax.experimental.pallas import tpu_sc as plsc

N = 10000
E = 320000
DIN = 128
H = 64
DE = 16
DEC = 32

NC = 2
NS = 16
CH = 128
EPT = E // (NC * NS)
NCHF = EPT // CH
TAIL = EPT - NCHF * CH
RZ = 1000
NZT = N // RZ

BLK = 2000
BLE = 4000

_MESH = plsc.VectorSubcoreMesh(core_axis_name="c", subcore_axis_name="s",
                               num_cores=NC, num_subcores=NS)


DW = 8


def _deg_body(dst1, zn, ones_h, out, d_v, d_t, ones_v, acc):
    c = lax.axis_index("c")
    s = lax.axis_index("s")
    w = c * NS + s
    ebase = w * EPT
    pltpu.sync_copy(ones_h, ones_v)

    @pl.when(s < NZT)
    def _zero():
        pltpu.sync_copy(zn.at[pl.ds(s * RZ, RZ)], acc.at[pl.ds(s * RZ, RZ)])

    plsc.subcore_barrier()

    def body(j, carry):
        pltpu.sync_copy(dst1.at[pl.ds(ebase + j * CH, CH)], d_v)
        pltpu.sync_copy(ones_v.at[pl.ds(0, CH)], acc.at[d_v], add=True)
        return carry

    lax.fori_loop(0, NCHF, body, 0)
    pltpu.sync_copy(dst1.at[pl.ds(ebase + NCHF * CH, TAIL)], d_t)
    pltpu.sync_copy(ones_v.at[pl.ds(0, TAIL)], acc.at[d_t], add=True)
    plsc.subcore_barrier()

    @pl.when(s < NZT)
    def _out():
        pltpu.sync_copy(acc.at[pl.ds(s * RZ, RZ)],
                        out.at[pl.ds(c * N + s * RZ, RZ)])


_deg_call = pl.kernel(
    _deg_body,
    out_type=jax.ShapeDtypeStruct((2 * N, DW), jnp.float32),
    mesh=_MESH,
    compiler_params=pltpu.CompilerParams(use_tc_tiling_on_sc=False),
    scratch_types=[
        pltpu.VMEM((CH,), jnp.int32),
        pltpu.VMEM((TAIL,), jnp.int32),
        pltpu.VMEM((CH, DW), jnp.float32),
        pltpu.VMEM_SHARED((N, DW), jnp.float32),
    ],
)


def _agg_body(hw, src1, dst1, z2d, out_a, out_b,
              s_v, d_v, s_t, d_t, rows, rows_t, acc, gsem):
    c = lax.axis_index("c")
    s = lax.axis_index("s")
    w = c * NS + s
    ebase = w * EPT

    @pl.when(s < NZT)
    def _zero():
        pltpu.sync_copy(z2d.at[pl.ds(s * RZ, RZ)], acc.at[pl.ds(s * RZ, RZ)])

    plsc.subcore_barrier()

    def body(j, carry):
        pltpu.sync_copy(src1.at[pl.ds(ebase + j * CH, CH)], s_v)
        pltpu.sync_copy(dst1.at[pl.ds(ebase + j * CH, CH)], d_v)
        pltpu.async_copy(hw.at[s_v], rows, gsem).wait()
        pltpu.sync_copy(rows, acc.at[d_v], add=True)
        return carry

    lax.fori_loop(0, NCHF, body, 0)
    pltpu.sync_copy(src1.at[pl.ds(ebase + NCHF * CH, TAIL)], s_t)
    pltpu.sync_copy(dst1.at[pl.ds(ebase + NCHF * CH, TAIL)], d_t)
    pltpu.async_copy(hw.at[s_t], rows_t, gsem).wait()
    pltpu.sync_copy(rows_t, acc.at[d_t], add=True)
    plsc.subcore_barrier()

    @pl.when((s < NZT) & (c == 0))
    def _out_a():
        pltpu.sync_copy(acc.at[pl.ds(s * RZ, RZ)],
                        out_a.at[pl.ds(s * RZ, RZ)])

    @pl.when((s < NZT) & (c == 1))
    def _out_b():
        pltpu.sync_copy(acc.at[pl.ds(s * RZ, RZ)],
                        out_b.at[pl.ds(s * RZ, RZ)])


_agg_call = pl.kernel(
    _agg_body,
    out_type=[jax.ShapeDtypeStruct((N, H), jnp.float32),
              jax.ShapeDtypeStruct((N, H), jnp.float32)],
    mesh=_MESH,
    compiler_params=pltpu.CompilerParams(use_tc_tiling_on_sc=False),
    scratch_types=[
        pltpu.VMEM((CH,), jnp.int32),
        pltpu.VMEM((CH,), jnp.int32),
        pltpu.VMEM((TAIL,), jnp.int32),
        pltpu.VMEM((TAIL,), jnp.int32),
        pltpu.VMEM((CH, H), jnp.float32),
        pltpu.VMEM((TAIL, H), jnp.float32),
        pltpu.VMEM_SHARED((N, H), jnp.float32),
        pltpu.SemaphoreType.DMA,
    ],
)


def _dec_body(p, q, src1, dst1, g1, g2,
              s_v, d_v, s_t, d_t, rp, rq, rp_t, rq_t, sp, sq):
    c = lax.axis_index("c")
    s = lax.axis_index("s")
    w = c * NS + s
    ebase = w * EPT

    def body(j, carry):
        pltpu.sync_copy(src1.at[pl.ds(ebase + j * CH, CH)], s_v)
        pltpu.sync_copy(dst1.at[pl.ds(ebase + j * CH, CH)], d_v)
        cp = pltpu.async_copy(p.at[s_v], rp, sp)
        cq = pltpu.async_copy(q.at[d_v], rq, sq)
        cp.wait()
        pltpu.sync_copy(rp, g1.at[pl.ds(ebase + j * CH, CH)])
        cq.wait()
        pltpu.sync_copy(rq, g2.at[pl.ds(ebase + j * CH, CH)])
        return carry

    lax.fori_loop(0, NCHF, body, 0)
    pltpu.sync_copy(src1.at[pl.ds(ebase + NCHF * CH, TAIL)], s_t)
    pltpu.sync_copy(dst1.at[pl.ds(ebase + NCHF * CH, TAIL)], d_t)
    cp = pltpu.async_copy(p.at[s_t], rp_t, sp)
    cq = pltpu.async_copy(q.at[d_t], rq_t, sq)
    cp.wait()
    pltpu.sync_copy(rp_t, g1.at[pl.ds(ebase + NCHF * CH, TAIL)])
    cq.wait()
    pltpu.sync_copy(rq_t, g2.at[pl.ds(ebase + NCHF * CH, TAIL)])


_dec_call = pl.kernel(
    _dec_body,
    out_type=[jax.ShapeDtypeStruct((E, DEC), jnp.float32),
              jax.ShapeDtypeStruct((E, DEC), jnp.float32)],
    mesh=_MESH,
    compiler_params=pltpu.CompilerParams(use_tc_tiling_on_sc=False),
    scratch_types=[
        pltpu.VMEM((CH,), jnp.int32),
        pltpu.VMEM((CH,), jnp.int32),
        pltpu.VMEM((TAIL,), jnp.int32),
        pltpu.VMEM((TAIL,), jnp.int32),
        pltpu.VMEM((CH, DEC), jnp.float32),
        pltpu.VMEM((CH, DEC), jnp.float32),
        pltpu.VMEM((TAIL, DEC), jnp.float32),
        pltpu.VMEM((TAIL, DEC), jnp.float32),
        pltpu.SemaphoreType.DMA,
        pltpu.SemaphoreType.DMA,
    ],
)


def _tc1_body(x_r, w0_r, dega_r, degb_r, hws_r, dinv_r):
    deg = dega_r[...] + degb_r[...] + 1.0
    dinv = lax.rsqrt(deg)
    hw = jnp.dot(x_r[...], w0_r[...], preferred_element_type=jnp.float32)
    hws_r[...] = hw * dinv
    dinv_r[...] = dinv


def _tc1(x, w0, dega, degb):
    return pl.pallas_call(
        _tc1_body,
        grid=(N // BLK,),
        in_specs=[
            pl.BlockSpec((BLK, DIN), lambda i: (i, 0)),
            pl.BlockSpec((DIN, H), lambda i: (0, 0)),
            pl.BlockSpec((BLK, 1), lambda i: (i, 0)),
            pl.BlockSpec((BLK, 1), lambda i: (i, 0)),
        ],
        out_specs=[
            pl.BlockSpec((BLK, H), lambda i: (i, 0)),
            pl.BlockSpec((BLK, 1), lambda i: (i, 0)),
        ],
        out_shape=[
            jax.ShapeDtypeStruct((N, H), jnp.float32),
            jax.ShapeDtypeStruct((N, 1), jnp.float32),
        ],
    )(x, w0, dega, degb)


def _prologue(agg_a, agg_b, hwp, dinv, b, g, be):
    pre = dinv * (agg_a + agg_b + hwp) + b
    mu = jnp.mean(pre, axis=-1, keepdims=True)
    d = pre - mu
    var = jnp.mean(d * d, axis=-1, keepdims=True)
    hn = d * lax.rsqrt(var + 1e-5) * g + be
    return jnp.maximum(hn, 0.0)


def _tc2_body(agg_a_r, agg_b_r, hwp_r, dinv_r, w_r, b_r, g_r, be_r, out_r):
    h = _prologue(agg_a_r[...], agg_b_r[...], hwp_r[...], dinv_r[...],
                  b_r[...], g_r[...], be_r[...])
    out_r[...] = jnp.dot(h, w_r[...],
                         preferred_element_type=jnp.float32) * dinv_r[...]


def _tc2(agg_a, agg_b, hwp, dinv, w, b, g, be):
    return pl.pallas_call(
        _tc2_body,
        grid=(N // BLK,),
        in_specs=[
            pl.BlockSpec((BLK, H), lambda i: (i, 0)),
            pl.BlockSpec((BLK, H), lambda i: (i, 0)),
            pl.BlockSpec((BLK, H), lambda i: (i, 0)),
            pl.BlockSpec((BLK, 1), lambda i: (i, 0)),
            pl.BlockSpec((H, H), lambda i: (0, 0)),
            pl.BlockSpec((1, H), lambda i: (0, 0)),
            pl.BlockSpec((1, H), lambda i: (0, 0)),
            pl.BlockSpec((1, H), lambda i: (0, 0)),
        ],
        out_specs=pl.BlockSpec((BLK, H), lambda i: (i, 0)),
        out_shape=jax.ShapeDtypeStruct((N, H), jnp.float32),
    )(agg_a, agg_b, hwp, dinv, w, b, g, be)


def _tc3_body(agg_a_r, agg_b_r, hwp_r, dinv_r, w1a_r, w1b_r, b_r, g_r, be_r,
              p_r, q_r):
    h = _prologue(agg_a_r[...], agg_b_r[...], hwp_r[...], dinv_r[...],
                  b_r[...], g_r[...], be_r[...])
    p_r[...] = jnp.dot(h, w1a_r[...], preferred_element_type=jnp.float32)
    q_r[...] = jnp.dot(h, w1b_r[...], preferred_element_type=jnp.float32)


def _tc3(agg_a, agg_b, hwp, dinv, w1a, w1b, b, g, be):
    return pl.pallas_call(
        _tc3_body,
        grid=(N // BLK,),
        in_specs=[
            pl.BlockSpec((BLK, H), lambda i: (i, 0)),
            pl.BlockSpec((BLK, H), lambda i: (i, 0)),
            pl.BlockSpec((BLK, H), lambda i: (i, 0)),
            pl.BlockSpec((BLK, 1), lambda i: (i, 0)),
            pl.BlockSpec((H, DEC), lambda i: (0, 0)),
            pl.BlockSpec((H, DEC), lambda i: (0, 0)),
            pl.BlockSpec((1, H), lambda i: (0, 0)),
            pl.BlockSpec((1, H), lambda i: (0, 0)),
            pl.BlockSpec((1, H), lambda i: (0, 0)),
        ],
        out_specs=[
            pl.BlockSpec((BLK, DEC), lambda i: (i, 0)),
            pl.BlockSpec((BLK, DEC), lambda i: (i, 0)),
        ],
        out_shape=[
            jax.ShapeDtypeStruct((N, DEC), jnp.float32),
            jax.ShapeDtypeStruct((N, DEC), jnp.float32),
        ],
    )(agg_a, agg_b, hwp, dinv, w1a, w1b, b, g, be)


E4 = E // 4
BL4 = BLE


def _tc4_body(g1_r, g2_r, ea_r, w1c_r, db1_r, w2_r, db2_r, w3_r, db3_r,
              out_r):
    r = jnp.dot(ea_r[...], w1c_r[...], preferred_element_type=jnp.float32)
    z = jnp.maximum(g1_r[...] + g2_r[...] + r + db1_r[...], 0.0)
    z2 = jnp.maximum(
        jnp.dot(z, w2_r[...], preferred_element_type=jnp.float32) + db2_r[...],
        0.0)
    out_r[...] = jnp.dot(z2, w3_r[...],
                         preferred_element_type=jnp.float32) + db3_r[...]


def _tc4(g1_4, g2_4, ea4, w1c4, db1_4, w2_4, db2_4, w3_4, db3_4):
    return pl.pallas_call(
        _tc4_body,
        grid=(E4 // BL4,),
        in_specs=[
            pl.BlockSpec((BL4, 128), lambda i: (i, 0)),
            pl.BlockSpec((BL4, 128), lambda i: (i, 0)),
            pl.BlockSpec((BL4, 4 * DE), lambda i: (i, 0)),
            pl.BlockSpec((4 * DE, 128), lambda i: (0, 0)),
            pl.BlockSpec((1, 128), lambda i: (0, 0)),
            pl.BlockSpec((128, 64), lambda i: (0, 0)),
            pl.BlockSpec((1, 64), lambda i: (0, 0)),
            pl.BlockSpec((64, 4), lambda i: (0, 0)),
            pl.BlockSpec((1, 4), lambda i: (0, 0)),
        ],
        out_specs=pl.BlockSpec((BL4, 4), lambda i: (i, 0)),
        out_shape=jax.ShapeDtypeStruct((E4, 4), jnp.float32),
    )(g1_4, g2_4, ea4, w1c4, db1_4, w2_4, db2_4, w3_4, db3_4)


def kernel(x, edge_index, edge_attr, W0, b0, W1, b1, W2, b2,
           g0, be0, g1, be1, g2, be2,
           dW1, db1, dW2, db2, dW3, db3):
    f32 = jnp.float32
    src = edge_index[0].astype(jnp.int32)
    dst = edge_index[1].astype(jnp.int32)
    zn = jnp.zeros((N, DW), f32)
    z2d = jnp.zeros((N, H), f32)
    ones_h = jnp.ones((CH, DW), f32)

    deg2 = _deg_call(dst, zn, ones_h)
    dega = deg2[:N, :1]
    degb = deg2[N:, :1]

    hws0, dinv = _tc1(x, W0, dega, degb)

    agg_a, agg_b = _agg_call(hws0, src, dst, z2d)
    hws1 = _tc2(agg_a, agg_b, hws0, dinv, W1,
                b0.reshape(1, H), g0.reshape(1, H), be0.reshape(1, H))

    agg_a, agg_b = _agg_call(hws1, src, dst, z2d)
    hws2 = _tc2(agg_a, agg_b, hws1, dinv, W2,
                b1.reshape(1, H), g1.reshape(1, H), be1.reshape(1, H))

    agg_a, agg_b = _agg_call(hws2, src, dst, z2d)
    p, q = _tc3(agg_a, agg_b, hws2, dinv, dW1[:H], dW1[H:2 * H],
                b2.reshape(1, H), g2.reshape(1, H), be2.reshape(1, H))

    g1e, g2e = _dec_call(p, q, src, dst)

    def bd4(w):
        din, dout = w.shape
        z = jnp.zeros((din, dout), f32)
        return jnp.concatenate([
            jnp.concatenate([w if i == j else z for j in range(4)], axis=1)
            for i in range(4)], axis=0)

    w1c4 = bd4(dW1[2 * H:])
    w2_4 = bd4(dW2)
    w3_4 = bd4(dW3)
    out4 = _tc4(g1e.reshape(E4, 128), g2e.reshape(E4, 128),
                edge_attr.reshape(E4, 4 * DE), w1c4,
                jnp.tile(db1, 4).reshape(1, 128),
                w2_4, jnp.tile(db2, 4).reshape(1, 64),
                w3_4, jnp.tile(db3, 4).reshape(1, 4))
    return out4.reshape(E)


# speedup vs baseline: 20.2254x; 1.5924x over previous
import jax
import jax.numpy as jnp
from jax import lax
from jax.experimental import pallas as pl
from jax.experimental.pallas import tpu as pltpu
from jax.experimental.pallas import tpu_sc as plsc

N = 10000
E = 320000
DIN = 128
H = 64
DE = 16
DEC = 32

NC = 2
NS = 16
CH = 128
EPT = E // (NC * NS)
NCHF = EPT // CH
TAIL = EPT - NCHF * CH
RZ = 1000
NZT = N // RZ

BLK = 2000
BLE = 4000

_MESH = plsc.VectorSubcoreMesh(core_axis_name="c", subcore_axis_name="s",
                               num_cores=NC, num_subcores=NS)


DW = 8


def _deg_body(dst1, zn, ones_h, out, d_v, d_t, ones_v, acc):
    c = lax.axis_index("c")
    s = lax.axis_index("s")
    w = c * NS + s
    ebase = w * EPT
    pltpu.sync_copy(ones_h, ones_v)

    @pl.when(s < NZT)
    def _zero():
        pltpu.sync_copy(zn.at[pl.ds(s * RZ, RZ)], acc.at[pl.ds(s * RZ, RZ)])

    plsc.subcore_barrier()

    def body(j, carry):
        pltpu.sync_copy(dst1.at[pl.ds(ebase + j * CH, CH)], d_v)
        pltpu.sync_copy(ones_v.at[pl.ds(0, CH)], acc.at[d_v], add=True)
        return carry

    lax.fori_loop(0, NCHF, body, 0)
    pltpu.sync_copy(dst1.at[pl.ds(ebase + NCHF * CH, TAIL)], d_t)
    pltpu.sync_copy(ones_v.at[pl.ds(0, TAIL)], acc.at[d_t], add=True)
    plsc.subcore_barrier()

    @pl.when(s < NZT)
    def _out():
        pltpu.sync_copy(acc.at[pl.ds(s * RZ, RZ)],
                        out.at[pl.ds(c * N + s * RZ, RZ)])


_deg_call = pl.kernel(
    _deg_body,
    out_type=jax.ShapeDtypeStruct((2 * N, DW), jnp.float32),
    mesh=_MESH,
    compiler_params=pltpu.CompilerParams(use_tc_tiling_on_sc=False),
    scratch_types=[
        pltpu.VMEM((CH,), jnp.int32),
        pltpu.VMEM((TAIL,), jnp.int32),
        pltpu.VMEM((CH, DW), jnp.float32),
        pltpu.VMEM_SHARED((N, DW), jnp.float32),
    ],
)


RING = 3
NIT3 = NCHF // RING


def _agg_body(hw, src1, dst1, z2d, out_a, out_b,
              s_flat, s_t, d_t, rows_t,
              d0, d1, d2, r0, r1, r2, acc,
              gs0, gs1, gs2, ds0, ds1, ds2, ss0, ss1, ss2):
    c = lax.axis_index("c")
    s = lax.axis_index("s")
    w = c * NS + s
    ebase = w * EPT
    d = (d0, d1, d2)
    r = (r0, r1, r2)
    gs = (gs0, gs1, gs2)
    ds = (ds0, ds1, ds2)
    ss = (ss0, ss1, ss2)

    @pl.when(s < NZT)
    def _zero():
        pltpu.sync_copy(z2d.at[pl.ds(s * RZ, RZ)], acc.at[pl.ds(s * RZ, RZ)])

    plsc.subcore_barrier()
    pltpu.sync_copy(src1.at[pl.ds(ebase, NCHF * CH)], s_flat)

    def start(j, b):
        pltpu.async_copy(dst1.at[pl.ds(ebase + j * CH, CH)], d[b], ds[b])
        pltpu.async_copy(hw.at[s_flat.at[pl.ds(j * CH, CH)]], r[b], gs[b])

    def wait_start(j, b):
        pltpu.make_async_copy(dst1.at[pl.ds(j * CH, CH)], d[b], ds[b]).wait()
        pltpu.make_async_copy(hw.at[s_flat.at[pl.ds(0, CH)]], r[b],
                              gs[b]).wait()

    def wait_scat(b):
        pltpu.make_async_copy(r[b], acc.at[d[b]], ss[b]).wait()

    for b in range(RING):
        start(b, b)

    def body(i, carry):
        for b in range(RING):
            j = RING * i + b
            wait_start(j, b)
            pltpu.async_copy(r[b], acc.at[d[b]], ss[b], add=True)

            @pl.when(j + RING < NCHF)
            def _pref():
                wait_scat(b)
                start(j + RING, b)
        return carry

    lax.fori_loop(0, NIT3, body, 0)
    for b in range(RING):
        wait_scat(b)
    pltpu.sync_copy(src1.at[pl.ds(ebase + NCHF * CH, TAIL)], s_t)
    pltpu.sync_copy(dst1.at[pl.ds(ebase + NCHF * CH, TAIL)], d_t)
    pltpu.async_copy(hw.at[s_t], rows_t, gs0).wait()
    pltpu.sync_copy(rows_t, acc.at[d_t], add=True)
    plsc.subcore_barrier()

    @pl.when((s < NZT) & (c == 0))
    def _out_a():
        pltpu.sync_copy(acc.at[pl.ds(s * RZ, RZ)],
                        out_a.at[pl.ds(s * RZ, RZ)])

    @pl.when((s < NZT) & (c == 1))
    def _out_b():
        pltpu.sync_copy(acc.at[pl.ds(s * RZ, RZ)],
                        out_b.at[pl.ds(s * RZ, RZ)])


_agg_call = pl.kernel(
    _agg_body,
    out_type=[jax.ShapeDtypeStruct((N, H), jnp.float32),
              jax.ShapeDtypeStruct((N, H), jnp.float32)],
    mesh=_MESH,
    compiler_params=pltpu.CompilerParams(use_tc_tiling_on_sc=False),
    scratch_types=[
        pltpu.VMEM((NCHF * CH,), jnp.int32),
        pltpu.VMEM((TAIL,), jnp.int32),
        pltpu.VMEM((TAIL,), jnp.int32),
        pltpu.VMEM((TAIL, H), jnp.float32),
        pltpu.VMEM((CH,), jnp.int32),
        pltpu.VMEM((CH,), jnp.int32),
        pltpu.VMEM((CH,), jnp.int32),
        pltpu.VMEM((CH, H), jnp.float32),
        pltpu.VMEM((CH, H), jnp.float32),
        pltpu.VMEM((CH, H), jnp.float32),
        pltpu.VMEM_SHARED((N, H), jnp.float32),
        pltpu.SemaphoreType.DMA,
        pltpu.SemaphoreType.DMA,
        pltpu.SemaphoreType.DMA,
        pltpu.SemaphoreType.DMA,
        pltpu.SemaphoreType.DMA,
        pltpu.SemaphoreType.DMA,
        pltpu.SemaphoreType.DMA,
        pltpu.SemaphoreType.DMA,
        pltpu.SemaphoreType.DMA,
    ],
)


def _dec_body(p, q, src1, dst1, g1, g2,
              s_v, d_v, s_t, d_t, rp, rq, rp_t, rq_t, sp, sq):
    c = lax.axis_index("c")
    s = lax.axis_index("s")
    w = c * NS + s
    ebase = w * EPT

    def body(j, carry):
        pltpu.sync_copy(src1.at[pl.ds(ebase + j * CH, CH)], s_v)
        pltpu.sync_copy(dst1.at[pl.ds(ebase + j * CH, CH)], d_v)
        cp = pltpu.async_copy(p.at[s_v], rp, sp)
        cq = pltpu.async_copy(q.at[d_v], rq, sq)
        cp.wait()
        pltpu.sync_copy(rp, g1.at[pl.ds(ebase + j * CH, CH)])
        cq.wait()
        pltpu.sync_copy(rq, g2.at[pl.ds(ebase + j * CH, CH)])
        return carry

    lax.fori_loop(0, NCHF, body, 0)
    pltpu.sync_copy(src1.at[pl.ds(ebase + NCHF * CH, TAIL)], s_t)
    pltpu.sync_copy(dst1.at[pl.ds(ebase + NCHF * CH, TAIL)], d_t)
    cp = pltpu.async_copy(p.at[s_t], rp_t, sp)
    cq = pltpu.async_copy(q.at[d_t], rq_t, sq)
    cp.wait()
    pltpu.sync_copy(rp_t, g1.at[pl.ds(ebase + NCHF * CH, TAIL)])
    cq.wait()
    pltpu.sync_copy(rq_t, g2.at[pl.ds(ebase + NCHF * CH, TAIL)])


_dec_call = pl.kernel(
    _dec_body,
    out_type=[jax.ShapeDtypeStruct((E, DEC), jnp.float32),
              jax.ShapeDtypeStruct((E, DEC), jnp.float32)],
    mesh=_MESH,
    compiler_params=pltpu.CompilerParams(use_tc_tiling_on_sc=False),
    scratch_types=[
        pltpu.VMEM((CH,), jnp.int32),
        pltpu.VMEM((CH,), jnp.int32),
        pltpu.VMEM((TAIL,), jnp.int32),
        pltpu.VMEM((TAIL,), jnp.int32),
        pltpu.VMEM((CH, DEC), jnp.float32),
        pltpu.VMEM((CH, DEC), jnp.float32),
        pltpu.VMEM((TAIL, DEC), jnp.float32),
        pltpu.VMEM((TAIL, DEC), jnp.float32),
        pltpu.SemaphoreType.DMA,
        pltpu.SemaphoreType.DMA,
    ],
)


def _tc1_body(x_r, w0_r, dega_r, degb_r, hws_r, dinv_r):
    deg = dega_r[...] + degb_r[...] + 1.0
    dinv = lax.rsqrt(deg)
    hw = jnp.dot(x_r[...], w0_r[...], preferred_element_type=jnp.float32)
    hws_r[...] = hw * dinv
    dinv_r[...] = dinv


def _tc1(x, w0, dega, degb):
    return pl.pallas_call(
        _tc1_body,
        grid=(N // BLK,),
        in_specs=[
            pl.BlockSpec((BLK, DIN), lambda i: (i, 0)),
            pl.BlockSpec((DIN, H), lambda i: (0, 0)),
            pl.BlockSpec((BLK, 1), lambda i: (i, 0)),
            pl.BlockSpec((BLK, 1), lambda i: (i, 0)),
        ],
        out_specs=[
            pl.BlockSpec((BLK, H), lambda i: (i, 0)),
            pl.BlockSpec((BLK, 1), lambda i: (i, 0)),
        ],
        out_shape=[
            jax.ShapeDtypeStruct((N, H), jnp.float32),
            jax.ShapeDtypeStruct((N, 1), jnp.float32),
        ],
    )(x, w0, dega, degb)


def _prologue(agg_a, agg_b, hwp, dinv, b, g, be):
    pre = dinv * (agg_a + agg_b + hwp) + b
    mu = jnp.mean(pre, axis=-1, keepdims=True)
    d = pre - mu
    var = jnp.mean(d * d, axis=-1, keepdims=True)
    hn = d * lax.rsqrt(var + 1e-5) * g + be
    return jnp.maximum(hn, 0.0)


def _tc2_body(agg_a_r, agg_b_r, hwp_r, dinv_r, w_r, b_r, g_r, be_r, out_r):
    h = _prologue(agg_a_r[...], agg_b_r[...], hwp_r[...], dinv_r[...],
                  b_r[...], g_r[...], be_r[...])
    out_r[...] = jnp.dot(h, w_r[...],
                         preferred_element_type=jnp.float32) * dinv_r[...]


def _tc2(agg_a, agg_b, hwp, dinv, w, b, g, be):
    return pl.pallas_call(
        _tc2_body,
        grid=(N // BLK,),
        in_specs=[
            pl.BlockSpec((BLK, H), lambda i: (i, 0)),
            pl.BlockSpec((BLK, H), lambda i: (i, 0)),
            pl.BlockSpec((BLK, H), lambda i: (i, 0)),
            pl.BlockSpec((BLK, 1), lambda i: (i, 0)),
            pl.BlockSpec((H, H), lambda i: (0, 0)),
            pl.BlockSpec((1, H), lambda i: (0, 0)),
            pl.BlockSpec((1, H), lambda i: (0, 0)),
            pl.BlockSpec((1, H), lambda i: (0, 0)),
        ],
        out_specs=pl.BlockSpec((BLK, H), lambda i: (i, 0)),
        out_shape=jax.ShapeDtypeStruct((N, H), jnp.float32),
    )(agg_a, agg_b, hwp, dinv, w, b, g, be)


def _tc3_body(agg_a_r, agg_b_r, hwp_r, dinv_r, w1a_r, w1b_r, b_r, g_r, be_r,
              p_r, q_r):
    h = _prologue(agg_a_r[...], agg_b_r[...], hwp_r[...], dinv_r[...],
                  b_r[...], g_r[...], be_r[...])
    p_r[...] = jnp.dot(h, w1a_r[...], preferred_element_type=jnp.float32)
    q_r[...] = jnp.dot(h, w1b_r[...], preferred_element_type=jnp.float32)


def _tc3(agg_a, agg_b, hwp, dinv, w1a, w1b, b, g, be):
    return pl.pallas_call(
        _tc3_body,
        grid=(N // BLK,),
        in_specs=[
            pl.BlockSpec((BLK, H), lambda i: (i, 0)),
            pl.BlockSpec((BLK, H), lambda i: (i, 0)),
            pl.BlockSpec((BLK, H), lambda i: (i, 0)),
            pl.BlockSpec((BLK, 1), lambda i: (i, 0)),
            pl.BlockSpec((H, DEC), lambda i: (0, 0)),
            pl.BlockSpec((H, DEC), lambda i: (0, 0)),
            pl.BlockSpec((1, H), lambda i: (0, 0)),
            pl.BlockSpec((1, H), lambda i: (0, 0)),
            pl.BlockSpec((1, H), lambda i: (0, 0)),
        ],
        out_specs=[
            pl.BlockSpec((BLK, DEC), lambda i: (i, 0)),
            pl.BlockSpec((BLK, DEC), lambda i: (i, 0)),
        ],
        out_shape=[
            jax.ShapeDtypeStruct((N, DEC), jnp.float32),
            jax.ShapeDtypeStruct((N, DEC), jnp.float32),
        ],
    )(agg_a, agg_b, hwp, dinv, w1a, w1b, b, g, be)


E4 = E // 4
BL4 = BLE


def _tc4_body(g1_r, g2_r, ea_r, w1c_r, db1_r, w2_r, db2_r, w3_r, db3_r,
              out_r):
    r = jnp.dot(ea_r[...], w1c_r[...], preferred_element_type=jnp.float32)
    z = jnp.maximum(g1_r[...] + g2_r[...] + r + db1_r[...], 0.0)
    z2 = jnp.maximum(
        jnp.dot(z, w2_r[...], preferred_element_type=jnp.float32) + db2_r[...],
        0.0)
    out_r[...] = jnp.dot(z2, w3_r[...],
                         preferred_element_type=jnp.float32) + db3_r[...]


def _tc4(g1_4, g2_4, ea4, w1c4, db1_4, w2_4, db2_4, w3_4, db3_4):
    return pl.pallas_call(
        _tc4_body,
        grid=(E4 // BL4,),
        in_specs=[
            pl.BlockSpec((BL4, 128), lambda i: (i, 0)),
            pl.BlockSpec((BL4, 128), lambda i: (i, 0)),
            pl.BlockSpec((BL4, 4 * DE), lambda i: (i, 0)),
            pl.BlockSpec((4 * DE, 128), lambda i: (0, 0)),
            pl.BlockSpec((1, 128), lambda i: (0, 0)),
            pl.BlockSpec((128, 64), lambda i: (0, 0)),
            pl.BlockSpec((1, 64), lambda i: (0, 0)),
            pl.BlockSpec((64, 4), lambda i: (0, 0)),
            pl.BlockSpec((1, 4), lambda i: (0, 0)),
        ],
        out_specs=pl.BlockSpec((BL4, 4), lambda i: (i, 0)),
        out_shape=jax.ShapeDtypeStruct((E4, 4), jnp.float32),
    )(g1_4, g2_4, ea4, w1c4, db1_4, w2_4, db2_4, w3_4, db3_4)


def kernel(x, edge_index, edge_attr, W0, b0, W1, b1, W2, b2,
           g0, be0, g1, be1, g2, be2,
           dW1, db1, dW2, db2, dW3, db3):
    f32 = jnp.float32
    src = edge_index[0].astype(jnp.int32)
    dst = edge_index[1].astype(jnp.int32)
    zn = jnp.zeros((N, DW), f32)
    z2d = jnp.zeros((N, H), f32)
    ones_h = jnp.ones((CH, DW), f32)

    deg2 = _deg_call(dst, zn, ones_h)
    dega = deg2[:N, :1]
    degb = deg2[N:, :1]

    hws0, dinv = _tc1(x, W0, dega, degb)

    agg_a, agg_b = _agg_call(hws0, src, dst, z2d)
    hws1 = _tc2(agg_a, agg_b, hws0, dinv, W1,
                b0.reshape(1, H), g0.reshape(1, H), be0.reshape(1, H))

    agg_a, agg_b = _agg_call(hws1, src, dst, z2d)
    hws2 = _tc2(agg_a, agg_b, hws1, dinv, W2,
                b1.reshape(1, H), g1.reshape(1, H), be1.reshape(1, H))

    agg_a, agg_b = _agg_call(hws2, src, dst, z2d)
    p, q = _tc3(agg_a, agg_b, hws2, dinv, dW1[:H], dW1[H:2 * H],
                b2.reshape(1, H), g2.reshape(1, H), be2.reshape(1, H))

    g1e, g2e = _dec_call(p, q, src, dst)

    def bd4(w):
        din, dout = w.shape
        z = jnp.zeros((din, dout), f32)
        return jnp.concatenate([
            jnp.concatenate([w if i == j else z for j in range(4)], axis=1)
            for i in range(4)], axis=0)

    w1c4 = bd4(dW1[2 * H:])
    w2_4 = bd4(dW2)
    w3_4 = bd4(dW3)
    out4 = _tc4(g1e.reshape(E4, 128), g2e.reshape(E4, 128),
                edge_attr.reshape(E4, 4 * DE), w1c4,
                jnp.tile(db1, 4).reshape(1, 128),
                w2_4, jnp.tile(db2, 4).reshape(1, 64),
                w3_4, jnp.tile(db3, 4).reshape(1, 4))
    return out4.reshape(E)


# speedup vs baseline: 23.1297x; 1.1436x over previous
import jax
import jax.numpy as jnp
from jax import lax
from jax.experimental import pallas as pl
from jax.experimental.pallas import tpu as pltpu
from jax.experimental.pallas import tpu_sc as plsc

N = 10000
E = 320000
DIN = 128
H = 64
DE = 16
DEC = 32

NC = 2
NS = 16
CH = 128
EPT = E // (NC * NS)
NCHF = EPT // CH
TAIL = EPT - NCHF * CH
RZ = 1000
NZT = N // RZ

BLK = 2000
BLE = 4000

_MESH = plsc.VectorSubcoreMesh(core_axis_name="c", subcore_axis_name="s",
                               num_cores=NC, num_subcores=NS)


DW = 8


def _deg_body(dst1, zn, ones_h, out, d_t, ones_v,
              d0, d1, d2, acc, ds0, ds1, ds2, ss0, ss1, ss2):
    c = lax.axis_index("c")
    s = lax.axis_index("s")
    w = c * NS + s
    ebase = w * EPT
    d = (d0, d1, d2)
    ds = (ds0, ds1, ds2)
    ss = (ss0, ss1, ss2)
    pltpu.sync_copy(ones_h, ones_v)

    @pl.when(s < NZT)
    def _zero():
        pltpu.sync_copy(zn.at[pl.ds(s * RZ, RZ)], acc.at[pl.ds(s * RZ, RZ)])

    plsc.subcore_barrier()

    def start(j, b):
        pltpu.async_copy(dst1.at[pl.ds(ebase + j * CH, CH)], d[b], ds[b])

    for b in range(RING):
        start(b, b)

    def body(i, carry):
        for b in range(RING):
            j = RING * i + b
            pltpu.make_async_copy(dst1.at[pl.ds(j * CH, CH)], d[b],
                                  ds[b]).wait()
            pltpu.async_copy(ones_v.at[pl.ds(0, CH)], acc.at[d[b]], ss[b],
                             add=True)

            @pl.when(j + RING < NCHF)
            def _pref():
                pltpu.make_async_copy(ones_v.at[pl.ds(0, CH)], acc.at[d[b]],
                                      ss[b]).wait()
                start(j + RING, b)
        return carry

    lax.fori_loop(0, NIT3, body, 0)
    for b in range(RING):
        pltpu.make_async_copy(ones_v.at[pl.ds(0, CH)], acc.at[d[b]],
                              ss[b]).wait()
    pltpu.sync_copy(dst1.at[pl.ds(ebase + NCHF * CH, TAIL)], d_t)
    pltpu.sync_copy(ones_v.at[pl.ds(0, TAIL)], acc.at[d_t], add=True)
    plsc.subcore_barrier()

    @pl.when(s < NZT)
    def _out():
        pltpu.sync_copy(acc.at[pl.ds(s * RZ, RZ)],
                        out.at[pl.ds(c * N + s * RZ, RZ)])


_deg_call = pl.kernel(
    _deg_body,
    out_type=jax.ShapeDtypeStruct((2 * N, DW), jnp.float32),
    mesh=_MESH,
    compiler_params=pltpu.CompilerParams(use_tc_tiling_on_sc=False),
    scratch_types=(
        [pltpu.VMEM((TAIL,), jnp.int32),
         pltpu.VMEM((CH, DW), jnp.float32)]
        + [pltpu.VMEM((CH,), jnp.int32)] * 3
        + [pltpu.VMEM_SHARED((N, DW), jnp.float32)]
        + [pltpu.SemaphoreType.DMA] * 6
    ),
)


RING = 3
NIT3 = NCHF // RING


def _agg_body(hw, src1, dst1, z2d, out_a, out_b,
              s_flat, s_t, d_t, rows_t,
              d0, d1, d2, r0, r1, r2, acc,
              gs0, gs1, gs2, ds0, ds1, ds2, ss0, ss1, ss2):
    c = lax.axis_index("c")
    s = lax.axis_index("s")
    w = c * NS + s
    ebase = w * EPT
    d = (d0, d1, d2)
    r = (r0, r1, r2)
    gs = (gs0, gs1, gs2)
    ds = (ds0, ds1, ds2)
    ss = (ss0, ss1, ss2)

    @pl.when(s < NZT)
    def _zero():
        pltpu.sync_copy(z2d.at[pl.ds(s * RZ, RZ)], acc.at[pl.ds(s * RZ, RZ)])

    plsc.subcore_barrier()
    pltpu.sync_copy(src1.at[pl.ds(ebase, NCHF * CH)], s_flat)

    def start(j, b):
        pltpu.async_copy(dst1.at[pl.ds(ebase + j * CH, CH)], d[b], ds[b])
        pltpu.async_copy(hw.at[s_flat.at[pl.ds(j * CH, CH)]], r[b], gs[b])

    def wait_start(j, b):
        pltpu.make_async_copy(dst1.at[pl.ds(j * CH, CH)], d[b], ds[b]).wait()
        pltpu.make_async_copy(hw.at[s_flat.at[pl.ds(0, CH)]], r[b],
                              gs[b]).wait()

    def wait_scat(b):
        pltpu.make_async_copy(r[b], acc.at[d[b]], ss[b]).wait()

    for b in range(RING):
        start(b, b)

    def body(i, carry):
        for b in range(RING):
            j = RING * i + b
            wait_start(j, b)
            pltpu.async_copy(r[b], acc.at[d[b]], ss[b], add=True)

            @pl.when(j + RING < NCHF)
            def _pref():
                wait_scat(b)
                start(j + RING, b)
        return carry

    lax.fori_loop(0, NIT3, body, 0)
    for b in range(RING):
        wait_scat(b)
    pltpu.sync_copy(src1.at[pl.ds(ebase + NCHF * CH, TAIL)], s_t)
    pltpu.sync_copy(dst1.at[pl.ds(ebase + NCHF * CH, TAIL)], d_t)
    pltpu.async_copy(hw.at[s_t], rows_t, gs0).wait()
    pltpu.sync_copy(rows_t, acc.at[d_t], add=True)
    plsc.subcore_barrier()

    @pl.when((s < NZT) & (c == 0))
    def _out_a():
        pltpu.sync_copy(acc.at[pl.ds(s * RZ, RZ)],
                        out_a.at[pl.ds(s * RZ, RZ)])

    @pl.when((s < NZT) & (c == 1))
    def _out_b():
        pltpu.sync_copy(acc.at[pl.ds(s * RZ, RZ)],
                        out_b.at[pl.ds(s * RZ, RZ)])


_agg_call = pl.kernel(
    _agg_body,
    out_type=[jax.ShapeDtypeStruct((N, H), jnp.float32),
              jax.ShapeDtypeStruct((N, H), jnp.float32)],
    mesh=_MESH,
    compiler_params=pltpu.CompilerParams(use_tc_tiling_on_sc=False),
    scratch_types=[
        pltpu.VMEM((NCHF * CH,), jnp.int32),
        pltpu.VMEM((TAIL,), jnp.int32),
        pltpu.VMEM((TAIL,), jnp.int32),
        pltpu.VMEM((TAIL, H), jnp.float32),
        pltpu.VMEM((CH,), jnp.int32),
        pltpu.VMEM((CH,), jnp.int32),
        pltpu.VMEM((CH,), jnp.int32),
        pltpu.VMEM((CH, H), jnp.float32),
        pltpu.VMEM((CH, H), jnp.float32),
        pltpu.VMEM((CH, H), jnp.float32),
        pltpu.VMEM_SHARED((N, H), jnp.float32),
        pltpu.SemaphoreType.DMA,
        pltpu.SemaphoreType.DMA,
        pltpu.SemaphoreType.DMA,
        pltpu.SemaphoreType.DMA,
        pltpu.SemaphoreType.DMA,
        pltpu.SemaphoreType.DMA,
        pltpu.SemaphoreType.DMA,
        pltpu.SemaphoreType.DMA,
        pltpu.SemaphoreType.DMA,
    ],
)


def _dec_body(p, q, src1, dst1, g1, g2,
              s_flat, d_flat, s_t, d_t, rp_t, rq_t,
              rp0, rp1, rp2, rq0, rq1, rq2,
              gp0, gp1, gp2, gq0, gq1, gq2,
              wp0, wp1, wp2, wq0, wq1, wq2):
    c = lax.axis_index("c")
    s = lax.axis_index("s")
    w = c * NS + s
    ebase = w * EPT
    rp = (rp0, rp1, rp2)
    rq = (rq0, rq1, rq2)
    gp = (gp0, gp1, gp2)
    gq = (gq0, gq1, gq2)
    wp = (wp0, wp1, wp2)
    wq = (wq0, wq1, wq2)
    pltpu.sync_copy(src1.at[pl.ds(ebase, NCHF * CH)], s_flat)
    pltpu.sync_copy(dst1.at[pl.ds(ebase, NCHF * CH)], d_flat)

    def start(j, b):
        pltpu.async_copy(p.at[s_flat.at[pl.ds(j * CH, CH)]], rp[b], gp[b])
        pltpu.async_copy(q.at[d_flat.at[pl.ds(j * CH, CH)]], rq[b], gq[b])

    for b in range(RING):
        start(b, b)

    def body(i, carry):
        for b in range(RING):
            j = RING * i + b
            pltpu.make_async_copy(p.at[s_flat.at[pl.ds(0, CH)]], rp[b],
                                  gp[b]).wait()
            pltpu.async_copy(rp[b], g1.at[pl.ds(ebase + j * CH, CH)], wp[b])
            pltpu.make_async_copy(q.at[d_flat.at[pl.ds(0, CH)]], rq[b],
                                  gq[b]).wait()
            pltpu.async_copy(rq[b], g2.at[pl.ds(ebase + j * CH, CH)], wq[b])

            @pl.when(j + RING < NCHF)
            def _pref():
                pltpu.make_async_copy(rp[b], g1.at[pl.ds(ebase, CH)],
                                      wp[b]).wait()
                pltpu.make_async_copy(rq[b], g2.at[pl.ds(ebase, CH)],
                                      wq[b]).wait()
                start(j + RING, b)
        return carry

    lax.fori_loop(0, NIT3, body, 0)
    for b in range(RING):
        pltpu.make_async_copy(rp[b], g1.at[pl.ds(ebase, CH)], wp[b]).wait()
        pltpu.make_async_copy(rq[b], g2.at[pl.ds(ebase, CH)], wq[b]).wait()
    pltpu.sync_copy(src1.at[pl.ds(ebase + NCHF * CH, TAIL)], s_t)
    pltpu.sync_copy(dst1.at[pl.ds(ebase + NCHF * CH, TAIL)], d_t)
    cp = pltpu.async_copy(p.at[s_t], rp_t, gp0)
    cq = pltpu.async_copy(q.at[d_t], rq_t, gq0)
    cp.wait()
    pltpu.sync_copy(rp_t, g1.at[pl.ds(ebase + NCHF * CH, TAIL)])
    cq.wait()
    pltpu.sync_copy(rq_t, g2.at[pl.ds(ebase + NCHF * CH, TAIL)])


_dec_call = pl.kernel(
    _dec_body,
    out_type=[jax.ShapeDtypeStruct((E, DEC), jnp.float32),
              jax.ShapeDtypeStruct((E, DEC), jnp.float32)],
    mesh=_MESH,
    compiler_params=pltpu.CompilerParams(use_tc_tiling_on_sc=False),
    scratch_types=(
        [pltpu.VMEM((NCHF * CH,), jnp.int32),
         pltpu.VMEM((NCHF * CH,), jnp.int32),
         pltpu.VMEM((TAIL,), jnp.int32),
         pltpu.VMEM((TAIL,), jnp.int32),
         pltpu.VMEM((TAIL, DEC), jnp.float32),
         pltpu.VMEM((TAIL, DEC), jnp.float32)]
        + [pltpu.VMEM((CH, DEC), jnp.float32)] * 6
        + [pltpu.SemaphoreType.DMA] * 12
    ),
)


def _tc1_body(x_r, w0_r, dega_r, degb_r, hws_r, dinv_r):
    deg = dega_r[...] + degb_r[...] + 1.0
    dinv = lax.rsqrt(deg)
    hw = jnp.dot(x_r[...], w0_r[...], preferred_element_type=jnp.float32)
    hws_r[...] = hw * dinv
    dinv_r[...] = dinv


def _tc1(x, w0, dega, degb):
    return pl.pallas_call(
        _tc1_body,
        grid=(N // BLK,),
        in_specs=[
            pl.BlockSpec((BLK, DIN), lambda i: (i, 0)),
            pl.BlockSpec((DIN, H), lambda i: (0, 0)),
            pl.BlockSpec((BLK, 1), lambda i: (i, 0)),
            pl.BlockSpec((BLK, 1), lambda i: (i, 0)),
        ],
        out_specs=[
            pl.BlockSpec((BLK, H), lambda i: (i, 0)),
            pl.BlockSpec((BLK, 1), lambda i: (i, 0)),
        ],
        out_shape=[
            jax.ShapeDtypeStruct((N, H), jnp.float32),
            jax.ShapeDtypeStruct((N, 1), jnp.float32),
        ],
    )(x, w0, dega, degb)


def _prologue(agg_a, agg_b, hwp, dinv, b, g, be):
    pre = dinv * (agg_a + agg_b + hwp) + b
    mu = jnp.mean(pre, axis=-1, keepdims=True)
    d = pre - mu
    var = jnp.mean(d * d, axis=-1, keepdims=True)
    hn = d * lax.rsqrt(var + 1e-5) * g + be
    return jnp.maximum(hn, 0.0)


def _tc2_body(agg_a_r, agg_b_r, hwp_r, dinv_r, w_r, b_r, g_r, be_r, out_r):
    h = _prologue(agg_a_r[...], agg_b_r[...], hwp_r[...], dinv_r[...],
                  b_r[...], g_r[...], be_r[...])
    out_r[...] = jnp.dot(h, w_r[...],
                         preferred_element_type=jnp.float32) * dinv_r[...]


def _tc2(agg_a, agg_b, hwp, dinv, w, b, g, be):
    return pl.pallas_call(
        _tc2_body,
        grid=(N // BLK,),
        in_specs=[
            pl.BlockSpec((BLK, H), lambda i: (i, 0)),
            pl.BlockSpec((BLK, H), lambda i: (i, 0)),
            pl.BlockSpec((BLK, H), lambda i: (i, 0)),
            pl.BlockSpec((BLK, 1), lambda i: (i, 0)),
            pl.BlockSpec((H, H), lambda i: (0, 0)),
            pl.BlockSpec((1, H), lambda i: (0, 0)),
            pl.BlockSpec((1, H), lambda i: (0, 0)),
            pl.BlockSpec((1, H), lambda i: (0, 0)),
        ],
        out_specs=pl.BlockSpec((BLK, H), lambda i: (i, 0)),
        out_shape=jax.ShapeDtypeStruct((N, H), jnp.float32),
    )(agg_a, agg_b, hwp, dinv, w, b, g, be)


def _tc3_body(agg_a_r, agg_b_r, hwp_r, dinv_r, w1a_r, w1b_r, b_r, g_r, be_r,
              p_r, q_r):
    h = _prologue(agg_a_r[...], agg_b_r[...], hwp_r[...], dinv_r[...],
                  b_r[...], g_r[...], be_r[...])
    p_r[...] = jnp.dot(h, w1a_r[...], preferred_element_type=jnp.float32)
    q_r[...] = jnp.dot(h, w1b_r[...], preferred_element_type=jnp.float32)


def _tc3(agg_a, agg_b, hwp, dinv, w1a, w1b, b, g, be):
    return pl.pallas_call(
        _tc3_body,
        grid=(N // BLK,),
        in_specs=[
            pl.BlockSpec((BLK, H), lambda i: (i, 0)),
            pl.BlockSpec((BLK, H), lambda i: (i, 0)),
            pl.BlockSpec((BLK, H), lambda i: (i, 0)),
            pl.BlockSpec((BLK, 1), lambda i: (i, 0)),
            pl.BlockSpec((H, DEC), lambda i: (0, 0)),
            pl.BlockSpec((H, DEC), lambda i: (0, 0)),
            pl.BlockSpec((1, H), lambda i: (0, 0)),
            pl.BlockSpec((1, H), lambda i: (0, 0)),
            pl.BlockSpec((1, H), lambda i: (0, 0)),
        ],
        out_specs=[
            pl.BlockSpec((BLK, DEC), lambda i: (i, 0)),
            pl.BlockSpec((BLK, DEC), lambda i: (i, 0)),
        ],
        out_shape=[
            jax.ShapeDtypeStruct((N, DEC), jnp.float32),
            jax.ShapeDtypeStruct((N, DEC), jnp.float32),
        ],
    )(agg_a, agg_b, hwp, dinv, w1a, w1b, b, g, be)


E4 = E // 4
BL4 = BLE


def _tc4_body(g1_r, g2_r, ea_r, w1c_r, db1_r, w2_r, db2_r, w3_r, db3_r,
              out_r):
    r = jnp.dot(ea_r[...], w1c_r[...], preferred_element_type=jnp.float32)
    z = jnp.maximum(g1_r[...] + g2_r[...] + r + db1_r[...], 0.0)
    z2 = jnp.maximum(
        jnp.dot(z, w2_r[...], preferred_element_type=jnp.float32) + db2_r[...],
        0.0)
    out_r[...] = jnp.dot(z2, w3_r[...],
                         preferred_element_type=jnp.float32) + db3_r[...]


def _tc4(g1_4, g2_4, ea4, w1c4, db1_4, w2_4, db2_4, w3_4, db3_4):
    return pl.pallas_call(
        _tc4_body,
        grid=(E4 // BL4,),
        in_specs=[
            pl.BlockSpec((BL4, 128), lambda i: (i, 0)),
            pl.BlockSpec((BL4, 128), lambda i: (i, 0)),
            pl.BlockSpec((BL4, 4 * DE), lambda i: (i, 0)),
            pl.BlockSpec((4 * DE, 128), lambda i: (0, 0)),
            pl.BlockSpec((1, 128), lambda i: (0, 0)),
            pl.BlockSpec((128, 64), lambda i: (0, 0)),
            pl.BlockSpec((1, 64), lambda i: (0, 0)),
            pl.BlockSpec((64, 4), lambda i: (0, 0)),
            pl.BlockSpec((1, 4), lambda i: (0, 0)),
        ],
        out_specs=pl.BlockSpec((BL4, 4), lambda i: (i, 0)),
        out_shape=jax.ShapeDtypeStruct((E4, 4), jnp.float32),
    )(g1_4, g2_4, ea4, w1c4, db1_4, w2_4, db2_4, w3_4, db3_4)


def kernel(x, edge_index, edge_attr, W0, b0, W1, b1, W2, b2,
           g0, be0, g1, be1, g2, be2,
           dW1, db1, dW2, db2, dW3, db3):
    f32 = jnp.float32
    src = edge_index[0].astype(jnp.int32)
    dst = edge_index[1].astype(jnp.int32)
    zn = jnp.zeros((N, DW), f32)
    z2d = jnp.zeros((N, H), f32)
    ones_h = jnp.ones((CH, DW), f32)

    deg2 = _deg_call(dst, zn, ones_h)
    dega = deg2[:N, :1]
    degb = deg2[N:, :1]

    hws0, dinv = _tc1(x, W0, dega, degb)

    agg_a, agg_b = _agg_call(hws0, src, dst, z2d)
    hws1 = _tc2(agg_a, agg_b, hws0, dinv, W1,
                b0.reshape(1, H), g0.reshape(1, H), be0.reshape(1, H))

    agg_a, agg_b = _agg_call(hws1, src, dst, z2d)
    hws2 = _tc2(agg_a, agg_b, hws1, dinv, W2,
                b1.reshape(1, H), g1.reshape(1, H), be1.reshape(1, H))

    agg_a, agg_b = _agg_call(hws2, src, dst, z2d)
    p, q = _tc3(agg_a, agg_b, hws2, dinv, dW1[:H], dW1[H:2 * H],
                b2.reshape(1, H), g2.reshape(1, H), be2.reshape(1, H))

    g1e, g2e = _dec_call(p, q, src, dst)

    def bd4(w):
        din, dout = w.shape
        z = jnp.zeros((din, dout), f32)
        return jnp.concatenate([
            jnp.concatenate([w if i == j else z for j in range(4)], axis=1)
            for i in range(4)], axis=0)

    w1c4 = bd4(dW1[2 * H:])
    w2_4 = bd4(dW2)
    w3_4 = bd4(dW3)
    out4 = _tc4(g1e.reshape(E4, 128), g2e.reshape(E4, 128),
                edge_attr.reshape(E4, 4 * DE), w1c4,
                jnp.tile(db1, 4).reshape(1, 128),
                w2_4, jnp.tile(db2, 4).reshape(1, 64),
                w3_4, jnp.tile(db3, 4).reshape(1, 4))
    return out4.reshape(E)
